# baseline (speedup 1.0000x reference)
import jax
import jax.numpy as jnp
from jax.experimental import pallas as pl
from jax.experimental.pallas import tpu as pltpu

B, S, H, D = 16, 2048, 8, 128
MAX_B, MAX_S = 16, 4096


def _copy_body(k_ref, v_ref, kc_ref, vc_ref, ok_ref, ov_ref,
               s0, s1, s2, s3):
    copies = []
    for b in range(MAX_B):
        copies.append(pltpu.make_async_copy(
            k_ref.at[b], ok_ref.at[b, 0:S], s0))
        copies.append(pltpu.make_async_copy(
            kc_ref.at[b, S:MAX_S], ok_ref.at[b, S:MAX_S], s1))
        copies.append(pltpu.make_async_copy(
            v_ref.at[b], ov_ref.at[b, 0:S], s2))
        copies.append(pltpu.make_async_copy(
            vc_ref.at[b, S:MAX_S], ov_ref.at[b, S:MAX_S], s3))
    for c in copies:
        c.start()
    for c in copies:
        c.wait()


def kernel(k, v, k_cache, v_cache):
    out_shape = jax.ShapeDtypeStruct((MAX_B, MAX_S, H, D), jnp.float32)
    hbm = pl.BlockSpec(memory_space=pltpu.MemorySpace.HBM)
    return pl.pallas_call(
        _copy_body,
        out_shape=(out_shape, out_shape),
        in_specs=[hbm, hbm, hbm, hbm],
        out_specs=(hbm, hbm),
        scratch_shapes=[pltpu.SemaphoreType.DMA] * 4,
    )(k, v, k_cache, v_cache)


# speedup vs baseline: 10.3842x; 10.3842x over previous
import jax
import jax.numpy as jnp
from jax.experimental import pallas as pl
from jax.experimental.pallas import tpu as pltpu

B, S, H, D = 16, 2048, 8, 128
MAX_B, MAX_S = 16, 4096
F = H * D
BS = 256
NS = S // BS


def _copy_body(k_ref, v_ref, kc_ref, vc_ref, ok_ref, ov_ref):
    h = pl.program_id(0)

    @pl.when(h == 0)
    def _():
        ok_ref[...] = k_ref[...]
        ov_ref[...] = v_ref[...]

    @pl.when(h == 1)
    def _():
        ok_ref[...] = kc_ref[...]
        ov_ref[...] = vc_ref[...]


def _src_map(h, b, s):
    return (jnp.where(h == 0, b, MAX_B - 1), 0,
            jnp.where(h == 0, s, NS - 1), 0)


def _cache_map(h, b, s):
    return (jnp.where(h == 1, b, 0), 1, jnp.where(h == 1, s, 0), 0)


def _out_map(h, b, s):
    return (b, h, s, 0)


def kernel(k, v, k_cache, v_cache):
    k4 = k.reshape(MAX_B, 1, S, F)
    v4 = v.reshape(MAX_B, 1, S, F)
    kc4 = k_cache.reshape(MAX_B, 2, S, F)
    vc4 = v_cache.reshape(MAX_B, 2, S, F)

    blk = (1, 1, BS, F)
    out_shape = jax.ShapeDtypeStruct((MAX_B, 2, S, F), jnp.float32)
    ok, ov = pl.pallas_call(
        _copy_body,
        grid=(2, MAX_B, NS),
        in_specs=[
            pl.BlockSpec(blk, _src_map),
            pl.BlockSpec(blk, _src_map),
            pl.BlockSpec(blk, _cache_map),
            pl.BlockSpec(blk, _cache_map),
        ],
        out_specs=(
            pl.BlockSpec(blk, _out_map),
            pl.BlockSpec(blk, _out_map),
        ),
        out_shape=(out_shape, out_shape),
        compiler_params=pltpu.CompilerParams(
            dimension_semantics=("arbitrary", "arbitrary", "arbitrary"),
        ),
    )(k4, v4, kc4, vc4)
    return (ok.reshape(MAX_B, MAX_S, H, D), ov.reshape(MAX_B, MAX_S, H, D))
